# baseline (speedup 1.0000x reference)
import functools
import numpy as np
import jax
import jax.numpy as jnp
from jax.experimental import pallas as pl
from jax.experimental.pallas import tpu as pltpu

K = 8
_KS1 = np.uint32(42)
_KS2 = np.uint32(0x1BD11BDA) ^ _KS1
_TINY = np.float32(np.finfo(np.float32).tiny)
_IMAX = np.int32(2**31 - 1)
_CH = 1024
_U = 12


def _rotl(v, r):
    return (v << jnp.uint32(r)) | (v >> jnp.uint32(32 - r))


def _threefry_bits(x1):
    rot_a = (13, 15, 26, 6)
    rot_b = (17, 29, 16, 24)
    x0 = x1
    x1 = _rotl(x1, 13) ^ x0
    for r in rot_a[1:]:
        x0 = x0 + x1
        x1 = _rotl(x1, r) ^ x0
    x0 = x0 + _KS1
    x1 = x1 + (_KS2 + np.uint32(1))
    inject = ((_KS2, np.uint32(2)), (None, _KS1 + np.uint32(3)),
              (_KS1, _KS2 + np.uint32(4)), (_KS2, np.uint32(5)))
    for g in range(4):
        for r in (rot_b if g % 2 == 0 else rot_a):
            x0 = x0 + x1
            x1 = _rotl(x1, r) ^ x0
        a, b = inject[g]
        if a is not None:
            x0 = x0 + a
        x1 = x1 + b
    return x0 ^ x1


def _sampler_kernel(x_ref, out_ref, *, vocab, n_loop, extra_bases, rpb):
    gi = pl.program_id(0)
    iota_k = (jax.lax.broadcasted_iota(jnp.uint32, (K, _CH), 0)
              * jnp.uint32(vocab)
              + jax.lax.broadcasted_iota(jnp.uint32, (K, _CH), 1))

    def row_body(row, _):
        cnt0 = ((jnp.uint32(gi * rpb) + row.astype(jnp.uint32))
                * jnp.uint32(K * vocab) + _KS1 + iota_k)

        def score_chunk(base_j):
            cnt = cnt0 + jnp.asarray(base_j).astype(jnp.uint32)
            bits = _threefry_bits(cnt)
            fb = (bits >> jnp.uint32(9)) | jnp.uint32(0x3F800000)
            f = (jax.lax.bitcast_convert_type(fb, jnp.float32)
                 - jnp.float32(1.0))
            u = f + _TINY
            g = -jnp.log(-jnp.log(u))
            xv = x_ref[row, pl.ds(base_j, _CH)].reshape(1, _CH)
            return g + jnp.log(jnp.maximum(xv, jnp.float32(1e-30)))

        def merge(carry, base_j):
            best_v, best_b = carry
            s = score_chunk(base_j)
            upd = s > best_v
            return (jnp.where(upd, s, best_v),
                    jnp.where(upd, jnp.asarray(base_j).astype(jnp.int32),
                              best_b))

        def body(c, carry):
            b0 = c * (_U * _CH)
            for t in range(_U):
                carry = merge(carry, b0 + t * _CH)
            return carry

        carry = (jnp.full((K, _CH), -jnp.inf, jnp.float32),
                 jnp.zeros((K, _CH), jnp.int32))
        carry = jax.lax.fori_loop(0, n_loop, body, carry)
        for b in extra_bases:
            carry = merge(carry, b)
        best_v, best_b = carry

        best_j = best_b + jax.lax.broadcasted_iota(jnp.int32, (K, _CH), 1)
        m = jnp.max(best_v, axis=1, keepdims=True)
        cand = jnp.where(best_v == m, best_j, _IMAX)
        idx = jnp.min(cand, axis=1, keepdims=True)
        out_ref[row, :, :] = jnp.broadcast_to(idx, (K, 128))
        return 0

    jax.lax.fori_loop(0, rpb, row_body, 0)


@jax.jit
def kernel(x):
    rows, vocab = x.shape
    rpb = 8 if rows % 8 == 0 else 1
    step = _U * _CH
    n_loop = vocab // step
    extra_bases = list(range(n_loop * step, vocab - _CH + 1, _CH))
    covered = n_loop * step + len(extra_bases) * _CH
    if covered < vocab:
        extra_bases.append(vocab - _CH)
    body = functools.partial(_sampler_kernel, vocab=vocab, n_loop=n_loop,
                             extra_bases=tuple(extra_bases), rpb=rpb)
    out = pl.pallas_call(
        body,
        grid=(rows // rpb,),
        in_specs=[pl.BlockSpec((rpb, vocab), lambda i: (i, 0))],
        out_specs=pl.BlockSpec((rpb, K, 128), lambda i: (i, 0, 0)),
        out_shape=jax.ShapeDtypeStruct((rows, K, 128), jnp.int32),
        compiler_params=pltpu.CompilerParams(
            dimension_semantics=("parallel",)),
    )(x)
    return out[:, :, 0]


# speedup vs baseline: 1.0040x; 1.0040x over previous
import functools
import numpy as np
import jax
import jax.numpy as jnp
from jax.experimental import pallas as pl
from jax.experimental.pallas import tpu as pltpu

K = 8
_KS1 = np.uint32(42)
_KS2 = np.uint32(0x1BD11BDA) ^ _KS1
_TINY = np.float32(np.finfo(np.float32).tiny)
_IMAX = np.int32(2**31 - 1)
_CH = 1024
_U = 48


def _rotl(v, r):
    return (v << jnp.uint32(r)) | (v >> jnp.uint32(32 - r))


def _threefry_bits(x1):
    rot_a = (13, 15, 26, 6)
    rot_b = (17, 29, 16, 24)
    x0 = x1
    x1 = _rotl(x1, 13) ^ x0
    for r in rot_a[1:]:
        x0 = x0 + x1
        x1 = _rotl(x1, r) ^ x0
    x0 = x0 + _KS1
    x1 = x1 + (_KS2 + np.uint32(1))
    inject = ((_KS2, np.uint32(2)), (None, _KS1 + np.uint32(3)),
              (_KS1, _KS2 + np.uint32(4)), (_KS2, np.uint32(5)))
    for g in range(4):
        for r in (rot_b if g % 2 == 0 else rot_a):
            x0 = x0 + x1
            x1 = _rotl(x1, r) ^ x0
        a, b = inject[g]
        if a is not None:
            x0 = x0 + a
        x1 = x1 + b
    return x0 ^ x1


def _sampler_kernel(x_ref, out_ref, *, vocab, n_loop, extra_bases, rpb):
    gi = pl.program_id(0)
    iota_k = (jax.lax.broadcasted_iota(jnp.uint32, (K, _CH), 0)
              * jnp.uint32(vocab)
              + jax.lax.broadcasted_iota(jnp.uint32, (K, _CH), 1))

    def row_body(row, _):
        cnt0 = ((jnp.uint32(gi * rpb) + row.astype(jnp.uint32))
                * jnp.uint32(K * vocab) + _KS1 + iota_k)

        def score_chunk(base_j):
            cnt = cnt0 + jnp.asarray(base_j).astype(jnp.uint32)
            bits = _threefry_bits(cnt)
            fb = (bits >> jnp.uint32(9)) | jnp.uint32(0x3F800000)
            f = (jax.lax.bitcast_convert_type(fb, jnp.float32)
                 - jnp.float32(1.0))
            u = f + _TINY
            g = -jnp.log(-jnp.log(u))
            xv = x_ref[row, pl.ds(base_j, _CH)].reshape(1, _CH)
            return g + jnp.log(jnp.maximum(xv, jnp.float32(1e-30)))

        def merge(carry, base_j):
            best_v, best_b = carry
            s = score_chunk(base_j)
            upd = s > best_v
            return (jnp.where(upd, s, best_v),
                    jnp.where(upd, jnp.asarray(base_j).astype(jnp.int32),
                              best_b))

        def body(c, carry):
            b0 = c * (_U * _CH)
            for t in range(_U):
                carry = merge(carry, b0 + t * _CH)
            return carry

        carry = (jnp.full((K, _CH), -jnp.inf, jnp.float32),
                 jnp.zeros((K, _CH), jnp.int32))
        carry = jax.lax.fori_loop(0, n_loop, body, carry)
        for b in extra_bases:
            carry = merge(carry, b)
        best_v, best_b = carry

        best_j = best_b + jax.lax.broadcasted_iota(jnp.int32, (K, _CH), 1)
        m = jnp.max(best_v, axis=1, keepdims=True)
        cand = jnp.where(best_v == m, best_j, _IMAX)
        idx = jnp.min(cand, axis=1, keepdims=True)
        out_ref[row, :, :] = jnp.broadcast_to(idx, (K, 128))
        return 0

    jax.lax.fori_loop(0, rpb, row_body, 0)


@jax.jit
def kernel(x):
    rows, vocab = x.shape
    rpb = 8 if rows % 8 == 0 else 1
    step = _U * _CH
    n_loop = vocab // step
    extra_bases = list(range(n_loop * step, vocab - _CH + 1, _CH))
    covered = n_loop * step + len(extra_bases) * _CH
    if covered < vocab:
        extra_bases.append(vocab - _CH)
    body = functools.partial(_sampler_kernel, vocab=vocab, n_loop=n_loop,
                             extra_bases=tuple(extra_bases), rpb=rpb)
    out = pl.pallas_call(
        body,
        grid=(rows // rpb,),
        in_specs=[pl.BlockSpec((rpb, vocab), lambda i: (i, 0))],
        out_specs=pl.BlockSpec((rpb, K, 128), lambda i: (i, 0, 0)),
        out_shape=jax.ShapeDtypeStruct((rows, K, 128), jnp.int32),
        compiler_params=pltpu.CompilerParams(
            dimension_semantics=("parallel",)),
    )(x)
    return out[:, :, 0]
